# Initial kernel scaffold; baseline (speedup 1.0000x reference)
import jax
import jax.numpy as jnp
from jax.experimental import pallas as pl


def kernel(x):
    raise NotImplementedError("write your pallas kernel here")


# speedup vs baseline: 6.5669x; 6.5669x over previous
import functools
import math

import jax
import jax.numpy as jnp
from jax import lax
from jax.experimental import pallas as pl
from jax.experimental.pallas import tpu as pltpu
from jax.experimental.pallas import tpu_sc as plsc

_L = 16
_NC = 2
_NS = 16


def _to_key(u):
    return u ^ ((u >> 31) & jnp.int32(0x7FFFFFFF))


def _popcount(mask):
    return jnp.max(plsc.all_reduce_population_count(mask))


def _vsort16(v, descending):
    k, _ = plsc.sort_key_val(v, v, descending=descending)
    return k


def _digit(key, shift):
    if shift == 24:
        return (key >> 24) + jnp.int32(128)
    if shift == 0:
        return key & jnp.int32(0xFF)
    return lax.shift_right_logical(key, jnp.int32(shift)) & jnp.int32(0xFF)


def _build(B, F, K, interpret=False):
    nv_row = F // _L
    nv_k = K // _L
    rows_per_w = B // (_NC * _NS)
    log2k = int(math.log2(K))

    mesh = plsc.VectorSubcoreMesh(
        core_axis_name="c", subcore_axis_name="s", num_cores=_NC, num_subcores=_NS
    )

    @functools.partial(
        pl.kernel,
        out_type=jax.ShapeDtypeStruct((B, K), jnp.float32),
        mesh=mesh,
        compiler_params=pltpu.CompilerParams(needs_layout_passes=False),
        scratch_types=[
            pltpu.VMEM((F,), jnp.float32),
            pltpu.VMEM((F + _L,), jnp.int32),
            pltpu.VMEM((F + _L,), jnp.int32),
            pltpu.VMEM((K + _L,), jnp.int32),
            pltpu.VMEM((_L * 256,), jnp.int32),
            pltpu.VMEM((256,), jnp.int32),
            pltpu.VMEM((K,), jnp.float32),
        ],
        interpret=interpret,
    )
    def topk_kernel(x_hbm, out_hbm, row_v, cand_a, cand_b, win, hist, hsum, outb):
        wid = lax.axis_index("s") * _NC + lax.axis_index("c")
        lanes = lax.iota(jnp.int32, _L)
        lane_base = lanes * jnp.int32(256)
        ones = jnp.ones((_L,), jnp.int32)
        zeros = jnp.zeros((_L,), jnp.int32)

        def clear_hist():
            def bd(i, _):
                hist[pl.ds(i * _L, _L)] = zeros
                return 0

            lax.fori_loop(0, 256, bd, 0)

        def hist_pass(src, n, shift, convert):
            trip = (n + _L - 1) // _L

            def bd(i, _):
                v = src[pl.ds(i * _L, _L)]
                key = _to_key(lax.bitcast_convert_type(v, jnp.int32)) if convert else v
                idx = lane_base + _digit(key, shift)
                lm = lanes < (n - i * _L)
                plsc.addupdate_scatter(hist, [idx], ones, mask=lm)
                return 0

            lax.fori_loop(0, trip, bd, 0)

        def reduce_hist():
            def bd(j, _):
                def inner(l, acc):
                    return acc + hist[pl.ds(l * 256 + j * _L, _L)]

                hsum[pl.ds(j * _L, _L)] = lax.fori_loop(
                    0, _L, inner, jnp.zeros((_L,), jnp.int32)
                )
                return 0

            lax.fori_loop(0, 16, bd, 0)

        def find_bin(rank):
            init = (
                jnp.bool_(False),
                jnp.int32(0),
                jnp.int32(0),
                jnp.int32(0),
            )

            def bd(m, carry):
                found, b_star, above, cnt_above = carry
                j = 15 - m
                h = hsum[pl.ds(j * _L, _L)]
                pre = plsc.cumsum(h)
                total = jnp.max(pre)
                suff = total - pre + h
                t = cnt_above + suff
                m_ge = t >= rank
                c = _popcount(m_ge)
                found_here = c > 0
                b_lane = c - 1
                lane_eq = lanes == b_lane
                t_b = jnp.max(jnp.where(lane_eq, t, 0))
                h_b = jnp.max(jnp.where(lane_eq, h, 0))
                use = jnp.logical_and(jnp.logical_not(found), found_here)
                b_star = jnp.where(use, j * _L + b_lane, b_star)
                above = jnp.where(use, t_b - h_b, above)
                found = jnp.logical_or(found, found_here)
                return found, b_star, above, cnt_above + total

            _, b_star, above, _ = lax.fori_loop(0, 16, bd, init)
            return b_star, above

        def compact(src, dst, n, shift, b_star, off_w, convert, store_keep):
            trip = (n + _L - 1) // _L

            def bd(i, carry):
                ow, ok = carry
                v = src[pl.ds(i * _L, _L)]
                key = _to_key(lax.bitcast_convert_type(v, jnp.int32)) if convert else v
                d = _digit(key, shift)
                lm = lanes < (n - i * _L)
                m_w = jnp.logical_and(d > b_star, lm)
                plsc.store_compressed(win.at[pl.ds(ow, _L)], key, mask=m_w)
                ow = ow + _popcount(m_w)
                if store_keep:
                    m_k = jnp.logical_and(d == b_star, lm)
                    plsc.store_compressed(dst.at[pl.ds(ok, _L)], key, mask=m_k)
                    ok = ok + _popcount(m_k)
                return ow, ok

            return lax.fori_loop(0, trip, bd, (off_w, jnp.int32(0)))

        def select_round(src, dst, n, shift, rank, off_w, convert, store_keep):
            clear_hist()
            hist_pass(src, n, shift, convert)
            reduce_hist()
            b_star, above, = find_bin(rank)
            off_w, n_keep = compact(
                src, dst, n, shift, b_star, off_w, convert, store_keep
            )
            return b_star, rank - above, off_w, n_keep

        def bitonic_sort():
            def init_bd(p, _):
                i0 = 2 * p * _L
                i1 = i0 + _L
                win[pl.ds(i0, _L)] = _vsort16(win[pl.ds(i0, _L)], True)
                win[pl.ds(i1, _L)] = _vsort16(win[pl.ds(i1, _L)], False)
                return 0

            lax.fori_loop(0, nv_k // 2, init_bd, 0)

            for k in range(5, log2k + 1):
                bs = 2 ** (k - 4)
                nblk = nv_k // bs
                n_even = (nblk + 1) // 2
                n_odd = nblk - n_even
                ppb = bs // 2

                dv = bs // 2
                while dv >= 1:
                    def mk_pair(desc, first_blk, dv=dv, bs=bs, ppb=ppb):
                        def bd(q, _):
                            blk = first_blk + 2 * (q // ppb)
                            w = q % ppb
                            i = blk * bs + (w // dv) * (2 * dv) + (w % dv)
                            j = i + dv
                            a = win[pl.ds(i * _L, _L)]
                            b = win[pl.ds(j * _L, _L)]
                            hi = jnp.maximum(a, b)
                            lo = jnp.minimum(a, b)
                            win[pl.ds(i * _L, _L)] = hi if desc else lo
                            win[pl.ds(j * _L, _L)] = lo if desc else hi
                            return 0

                        return bd

                    lax.fori_loop(0, n_even * ppb, mk_pair(True, 0), 0)
                    if n_odd:
                        lax.fori_loop(0, n_odd * ppb, mk_pair(False, 1), 0)
                    dv //= 2

                def mk_sort(desc, first_blk, bs=bs):
                    def bd(q, _):
                        blk = first_blk + 2 * (q // bs)
                        i = (blk * bs + q % bs) * _L
                        win[pl.ds(i, _L)] = _vsort16(win[pl.ds(i, _L)], desc)
                        return 0

                    return bd

                lax.fori_loop(0, n_even * bs, mk_sort(True, 0), 0)
                if n_odd:
                    lax.fori_loop(0, n_odd * bs, mk_sort(False, 1), 0)

        def do_row(r, _):
            row = wid * rows_per_w + r
            pltpu.sync_copy(x_hbm.at[row], row_v)

            rank = jnp.int32(K)
            off_w = jnp.int32(0)
            b1, rank, off_w, nc = select_round(
                row_v, cand_a, F, 24, rank, off_w, True, True
            )
            b2, rank, off_w, nc = select_round(
                cand_a, cand_b, nc, 16, rank, off_w, False, True
            )
            b3, rank, off_w, nc = select_round(
                cand_b, cand_a, nc, 8, rank, off_w, False, True
            )
            b4, rank, off_w, _ = select_round(
                cand_a, cand_b, nc, 0, rank, off_w, False, False
            )
            kstar = ((b1 ^ 128) << 24) | (b2 << 16) | (b3 << 8) | b4
            kstar_v = jnp.broadcast_to(kstar, (_L,))

            def app_bd(i, carry):
                off, rem = carry
                m = lanes < rem
                plsc.store_compressed(win.at[pl.ds(off, _L)], kstar_v, mask=m)
                c = jnp.minimum(rem, _L)
                return off + c, rem - c

            lax.fori_loop(0, (rank + _L - 1) // _L, app_bd, (off_w, rank))

            bitonic_sort()

            def out_bd(i, _):
                kv = win[pl.ds(i * _L, _L)]
                outb[pl.ds(i * _L, _L)] = lax.bitcast_convert_type(
                    _to_key(kv), jnp.float32
                )
                return 0

            lax.fori_loop(0, nv_k, out_bd, 0)
            pltpu.sync_copy(outb, out_hbm.at[row])
            return 0

        lax.fori_loop(0, rows_per_w, do_row, 0)

    return topk_kernel


_topk = _build(64, 32768, 1024)


def kernel(x):
    return _topk(x)[:, :, None]


# speedup vs baseline: 6.6267x; 1.0091x over previous
import functools
import math

import jax
import jax.numpy as jnp
from jax import lax
from jax.experimental import pallas as pl
from jax.experimental.pallas import tpu as pltpu
from jax.experimental.pallas import tpu_sc as plsc

_L = 16
_NC = 2
_NS = 16


def _to_key(u):
    return u ^ ((u >> 31) & jnp.int32(0x7FFFFFFF))


def _popcount(mask):
    return jnp.max(plsc.all_reduce_population_count(mask))


def _vsort16(v, descending):
    k, _ = plsc.sort_key_val(v, v, descending=descending)
    return k


def _digit(key, shift):
    if shift == 24:
        return (key >> 24) + jnp.int32(128)
    if shift == 0:
        return key & jnp.int32(0xFF)
    return lax.shift_right_logical(key, jnp.int32(shift)) & jnp.int32(0xFF)


def _build(B, F, K, interpret=False):
    nv_row = F // _L
    nv_k = K // _L
    rows_per_w = B // (_NC * _NS)
    log2k = int(math.log2(K))

    mesh = plsc.VectorSubcoreMesh(
        core_axis_name="c", subcore_axis_name="s", num_cores=_NC, num_subcores=_NS
    )

    @functools.partial(
        pl.kernel,
        out_type=jax.ShapeDtypeStruct((B, K), jnp.float32),
        mesh=mesh,
        compiler_params=pltpu.CompilerParams(needs_layout_passes=False),
        scratch_types=[
            pltpu.VMEM((F,), jnp.float32),
            pltpu.VMEM((F + 4 * _L,), jnp.int32),
            pltpu.VMEM((F + 4 * _L,), jnp.int32),
            pltpu.VMEM((K + _L,), jnp.int32),
            pltpu.VMEM((_L * 256,), jnp.int32),
            pltpu.VMEM((256,), jnp.int32),
            pltpu.VMEM((K,), jnp.float32),
        ],
        interpret=interpret,
    )
    def topk_kernel(x_hbm, out_hbm, row_v, cand_a, cand_b, win, hist, hsum, outb):
        wid = lax.axis_index("s") * _NC + lax.axis_index("c")
        lanes = lax.iota(jnp.int32, _L)
        lane_base = lanes * jnp.int32(256)
        ones = jnp.ones((_L,), jnp.int32)
        zeros = jnp.zeros((_L,), jnp.int32)

        def clear_hist():
            def bd(i, _):
                hist[pl.ds(i * _L, _L)] = zeros
                return 0

            lax.fori_loop(0, 256, bd, 0)

        def hist_pass(src, n, shift, convert):
            static = isinstance(n, int)
            trip = (n + _L - 1) // _L

            def bd(i, _):
                v = src[pl.ds(i * _L, _L)]
                key = _to_key(lax.bitcast_convert_type(v, jnp.int32)) if convert else v
                idx = lane_base + _digit(key, shift)
                if static:
                    plsc.addupdate_scatter(hist, [idx], ones)
                else:
                    lm = lanes < (n - i * _L)
                    plsc.addupdate_scatter(hist, [idx], ones, mask=lm)
                return 0

            lax.fori_loop(0, trip, bd, 0)

        def reduce_hist():
            def bd(j, _):
                def inner(l, acc):
                    return acc + hist[pl.ds(l * 256 + j * _L, _L)]

                hsum[pl.ds(j * _L, _L)] = lax.fori_loop(
                    0, _L, inner, jnp.zeros((_L,), jnp.int32)
                )
                return 0

            lax.fori_loop(0, 16, bd, 0)

        def find_bin(rank):
            init = (
                jnp.bool_(False),
                jnp.int32(0),
                jnp.int32(0),
                jnp.int32(0),
            )

            def bd(m, carry):
                found, b_star, above, cnt_above = carry
                j = 15 - m
                h = hsum[pl.ds(j * _L, _L)]
                pre = plsc.cumsum(h)
                total = jnp.max(pre)
                suff = total - pre + h
                t = cnt_above + suff
                m_ge = t >= rank
                c = _popcount(m_ge)
                found_here = c > 0
                b_lane = c - 1
                lane_eq = lanes == b_lane
                t_b = jnp.max(jnp.where(lane_eq, t, 0))
                h_b = jnp.max(jnp.where(lane_eq, h, 0))
                use = jnp.logical_and(jnp.logical_not(found), found_here)
                b_star = jnp.where(use, j * _L + b_lane, b_star)
                above = jnp.where(use, t_b - h_b, above)
                found = jnp.logical_or(found, found_here)
                return found, b_star, above, cnt_above + total

            _, b_star, above, _ = lax.fori_loop(0, 16, bd, init)
            return b_star, above

        def compact(src, dst, n, shift, b_star, off_w, convert, store_keep):
            static = isinstance(n, int)
            trip = (n + _L - 1) // _L
            trip2 = (trip + 1) // 2

            def one(i, ow, ok):
                v = src[pl.ds(i * _L, _L)]
                key = _to_key(lax.bitcast_convert_type(v, jnp.int32)) if convert else v
                d = _digit(key, shift)
                m_w = d > b_star
                m_k = d == b_star
                if not static:
                    lm = lanes < (n - i * _L)
                    m_w = jnp.logical_and(m_w, lm)
                    m_k = jnp.logical_and(m_k, lm)
                plsc.store_compressed(win.at[pl.ds(ow, _L)], key, mask=m_w)
                ow = ow + _popcount(m_w)
                if store_keep:
                    plsc.store_compressed(dst.at[pl.ds(ok, _L)], key, mask=m_k)
                    ok = ok + _popcount(m_k)
                return ow, ok

            def bd(j, carry):
                ow, ok = carry
                return one(j, ow, ok)

            return lax.fori_loop(0, trip, bd, (off_w, jnp.int32(0)))

        def select_round(src, dst, n, shift, rank, off_w, convert, store_keep):
            clear_hist()
            hist_pass(src, n, shift, convert)
            reduce_hist()
            b_star, above, = find_bin(rank)
            off_w, n_keep = compact(
                src, dst, n, shift, b_star, off_w, convert, store_keep
            )
            return b_star, rank - above, off_w, n_keep

        def bitonic_sort():
            def init_bd(p, _):
                i0 = 2 * p * _L
                i1 = i0 + _L
                win[pl.ds(i0, _L)] = _vsort16(win[pl.ds(i0, _L)], True)
                win[pl.ds(i1, _L)] = _vsort16(win[pl.ds(i1, _L)], False)
                return 0

            lax.fori_loop(0, nv_k // 2, init_bd, 0)

            for k in range(5, log2k + 1):
                bs = 2 ** (k - 4)
                nblk = nv_k // bs
                n_even = (nblk + 1) // 2
                n_odd = nblk - n_even
                ppb = bs // 2

                dv = bs // 2
                while dv >= 1:
                    def run_pairs(desc, first_blk, count, dv=dv, bs=bs, ppb=ppb):
                        def bd(q, _):
                            blk = first_blk + 2 * (q // ppb)
                            w = q % ppb
                            i = blk * bs + (w // dv) * (2 * dv) + (w % dv)
                            j = i + dv
                            a = win[pl.ds(i * _L, _L)]
                            b = win[pl.ds(j * _L, _L)]
                            hi = jnp.maximum(a, b)
                            lo = jnp.minimum(a, b)
                            win[pl.ds(i * _L, _L)] = hi if desc else lo
                            win[pl.ds(j * _L, _L)] = lo if desc else hi
                            return 0

                        lax.fori_loop(0, count, bd, 0)

                    run_pairs(True, 0, n_even * ppb)
                    if n_odd:
                        run_pairs(False, 1, n_odd * ppb)
                    dv //= 2

                def run_sorts(desc, first_blk, count, bs=bs):
                    def bd(q, _):
                        blk = first_blk + 2 * (q // bs)
                        i = (blk * bs + q % bs) * _L
                        win[pl.ds(i, _L)] = _vsort16(win[pl.ds(i, _L)], desc)
                        return 0

                    lax.fori_loop(0, count, bd, 0)

                run_sorts(True, 0, n_even * bs)
                if n_odd:
                    run_sorts(False, 1, n_odd * bs)

        def do_row(r, _):
            row = wid * rows_per_w + r
            pltpu.sync_copy(x_hbm.at[row], row_v)

            rank = jnp.int32(K)
            off_w = jnp.int32(0)
            b1, rank, off_w, nc = select_round(
                row_v, cand_a, F, 24, rank, off_w, True, True
            )
            b2, rank, off_w, nc = select_round(
                cand_a, cand_b, nc, 16, rank, off_w, False, True
            )
            b3, rank, off_w, nc = select_round(
                cand_b, cand_a, nc, 8, rank, off_w, False, True
            )
            b4, rank, off_w, _ = select_round(
                cand_a, cand_b, nc, 0, rank, off_w, False, False
            )
            kstar = ((b1 ^ 128) << 24) | (b2 << 16) | (b3 << 8) | b4
            kstar_v = jnp.broadcast_to(kstar, (_L,))

            def app_bd(i, carry):
                off, rem = carry
                m = lanes < rem
                plsc.store_compressed(win.at[pl.ds(off, _L)], kstar_v, mask=m)
                c = jnp.minimum(rem, _L)
                return off + c, rem - c

            lax.fori_loop(0, (rank + _L - 1) // _L, app_bd, (off_w, rank))

            bitonic_sort()

            def out_bd(i, _):
                kv = win[pl.ds(i * _L, _L)]
                outb[pl.ds(i * _L, _L)] = lax.bitcast_convert_type(
                    _to_key(kv), jnp.float32
                )
                return 0

            lax.fori_loop(0, nv_k, out_bd, 0)
            pltpu.sync_copy(outb, out_hbm.at[row])
            return 0

        lax.fori_loop(0, rows_per_w, do_row, 0)

    return topk_kernel


_topk = _build(64, 32768, 1024)


def kernel(x):
    return _topk(x)[:, :, None]
